# baseline (speedup 1.0000x reference)
import dataclasses
import functools

import jax
import jax.numpy as jnp
from jax import lax
from jax.experimental import pallas as pl
from jax.experimental.pallas import tpu as pltpu
from jax.experimental.pallas import tpu_sc as plsc

TOP_K = 8
E = 64
L = 16
ROWS_PER_BLOCK = 128
N_WORKERS = 32
BLOCKS_PER_WORKER = 8


def _merge16(k0, p0, k1, p1):
    rk = lax.rev(k1, (0,))
    rp = lax.rev(p1, (0,))
    take0 = k0 >= rk
    km = jnp.where(take0, k0, rk)
    pm = jnp.where(take0, p0, rp)
    return plsc.sort_key_val(km, pm, descending=True)


def _compute_block(x_vmem, sw_ref, si_ref):
    iota = lax.iota(jnp.int32, L)
    idx_base = [iota + L * j for j in range(4)]

    @pl.loop(0, ROWS_PER_BLOCK)
    def _(row):
        xrow = x_vmem.at[row]
        chunks = [xrow[pl.ds(L * j, L)] for j in range(4)]
        sorted_kp = [
            plsc.sort_key_val(chunks[j], idx_base[j], descending=True)
            for j in range(4)
        ]
        k01, p01 = _merge16(*sorted_kp[0], *sorted_kp[1])
        k23, p23 = _merge16(*sorted_kp[2], *sorted_kp[3])
        kf, pf = _merge16(k01, p01, k23, p23)
        e_sum = (
            jnp.exp(chunks[0])
            + jnp.exp(chunks[1])
            + jnp.exp(chunks[2])
            + jnp.exp(chunks[3])
        )
        s = jnp.sum(e_sum)
        s_vec = lax.broadcast_in_dim(s, (L,), ())
        sw_ref.at[row][...] = jnp.exp(kf) / s_vec
        si_ref.at[row][...] = pf


@jax.jit
def kernel(x):
    n, e = x.shape
    mesh = plsc.VectorSubcoreMesh(core_axis_name="c", subcore_axis_name="s")
    cp = pltpu.CompilerParams()
    fields = pltpu.CompilerParams.__dataclass_fields__
    if "needs_layout_passes" in fields:
        cp = dataclasses.replace(cp, needs_layout_passes=False)
    if "use_tc_tiling_on_sc" in fields:
        cp = dataclasses.replace(cp, use_tc_tiling_on_sc=False)

    @functools.partial(
        pl.kernel,
        out_type=(
            jax.ShapeDtypeStruct((n, TOP_K), jnp.float32),
            jax.ShapeDtypeStruct((n, TOP_K), jnp.int32),
        ),
        mesh=mesh,
        compiler_params=cp,
        scratch_types=[
            pltpu.VMEM((ROWS_PER_BLOCK, E), jnp.float32),
            pltpu.VMEM((ROWS_PER_BLOCK, E), jnp.float32),
            pltpu.VMEM((ROWS_PER_BLOCK, L), jnp.float32),
            pltpu.VMEM((ROWS_PER_BLOCK, L), jnp.float32),
            pltpu.VMEM((ROWS_PER_BLOCK, L), jnp.int32),
            pltpu.VMEM((ROWS_PER_BLOCK, L), jnp.int32),
            pltpu.SemaphoreType.DMA,
            pltpu.SemaphoreType.DMA,
            pltpu.SemaphoreType.DMA,
            pltpu.SemaphoreType.DMA,
            pltpu.SemaphoreType.DMA,
            pltpu.SemaphoreType.DMA,
        ],
    )
    def sc_run(x_hbm, w_hbm, i_hbm, xb0, xb1, sw0, sw1, si0, si1,
               sx0, sx1, sw_sem0, sw_sem1, si_sem0, si_sem1):
        wid = lax.axis_index("c") * 16 + lax.axis_index("s")
        base = wid * BLOCKS_PER_WORKER
        xbufs, xsems = [xb0, xb1], [sx0, sx1]
        swbufs, wsems = [sw0, sw1], [sw_sem0, sw_sem1]
        sibufs, isems = [si0, si1], [si_sem0, si_sem1]

        def in_copy(b, k):
            rows = (base + b) * ROWS_PER_BLOCK
            return pltpu.make_async_copy(
                x_hbm.at[pl.ds(rows, ROWS_PER_BLOCK), :], xbufs[k], xsems[k]
            )

        def w_copy(k, b):
            rows = (base + b) * ROWS_PER_BLOCK
            return pltpu.make_async_copy(
                swbufs[k].at[:, pl.ds(0, TOP_K)],
                w_hbm.at[pl.ds(rows, ROWS_PER_BLOCK), :],
                wsems[k],
            )

        def i_copy(k, b):
            rows = (base + b) * ROWS_PER_BLOCK
            return pltpu.make_async_copy(
                sibufs[k].at[:, pl.ds(0, TOP_K)],
                i_hbm.at[pl.ds(rows, ROWS_PER_BLOCK), :],
                isems[k],
            )

        in_copy(0, 0).start()
        for b in range(BLOCKS_PER_WORKER):
            cur, nxt = b % 2, (b + 1) % 2
            if b + 1 < BLOCKS_PER_WORKER:
                in_copy(b + 1, nxt).start()
            in_copy(b, cur).wait()
            if b >= 2:
                w_copy(cur, b - 2).wait()
                i_copy(cur, b - 2).wait()
            _compute_block(xbufs[cur], swbufs[cur], sibufs[cur])
            w_copy(cur, b).start()
            i_copy(cur, b).start()
        for b in (BLOCKS_PER_WORKER - 2, BLOCKS_PER_WORKER - 1):
            w_copy(b % 2, b).wait()
            i_copy(b % 2, b).wait()

    return sc_run(x)


# speedup vs baseline: 1.4588x; 1.4588x over previous
import dataclasses
import functools

import jax
import jax.numpy as jnp
from jax import lax
from jax.experimental import pallas as pl
from jax.experimental.pallas import tpu as pltpu
from jax.experimental.pallas import tpu_sc as plsc

TOP_K = 8
E = 64
L = 16
ROWS_PER_BLOCK = 128


def _merge16(k0, p0, k1, p1):
    rk = lax.rev(k1, (0,))
    rp = lax.rev(p1, (0,))
    take0 = k0 >= rk
    km = jnp.where(take0, k0, rk)
    pm = jnp.where(take0, p0, rp)
    return plsc.sort_key_val(km, pm, descending=True)


def _sc_body(x_vmem, w_vmem, i_vmem):
    iota = lax.iota(jnp.int32, L)
    idx_base = [iota + L * j for j in range(4)]

    @pl.loop(0, ROWS_PER_BLOCK)
    def _(row):
        xrow = x_vmem.at[row]
        chunks = [xrow[pl.ds(L * j, L)] for j in range(4)]
        sorted_kp = [
            plsc.sort_key_val(chunks[j], idx_base[j], descending=True)
            for j in range(4)
        ]
        k01, p01 = _merge16(*sorted_kp[0], *sorted_kp[1])
        k23, p23 = _merge16(*sorted_kp[2], *sorted_kp[3])
        kf, pf = _merge16(k01, p01, k23, p23)
        e_sum = (
            jnp.exp(chunks[0])
            + jnp.exp(chunks[1])
            + jnp.exp(chunks[2])
            + jnp.exp(chunks[3])
        )
        s = jnp.sum(e_sum)
        s_vec = lax.broadcast_in_dim(s, (L,), ())
        w_vmem[row, pl.ds(0, L)] = jnp.exp(kf) / s_vec
        i_vmem[row, pl.ds(0, L)] = pf


@jax.jit
def kernel(x):
    n, e = x.shape
    n_blocks = n // ROWS_PER_BLOCK
    mesh = plsc.VectorSubcoreMesh(core_axis_name="c", subcore_axis_name="s")
    cp = pltpu.CompilerParams()
    if "needs_layout_passes" in pltpu.CompilerParams.__dataclass_fields__:
        cp = dataclasses.replace(cp, needs_layout_passes=False)

    @functools.partial(
        pl.kernel,
        out_type=(
            jax.ShapeDtypeStruct((n, 128), jnp.float32),
            jax.ShapeDtypeStruct((n, 128), jnp.int32),
        ),
        mesh=mesh,
        compiler_params=cp,
    )
    def sc_run(x_hbm, w_hbm, i_hbm):
        pltpu.emit_pipeline(
            _sc_body,
            grid=(n_blocks,),
            in_specs=[
                pl.BlockSpec((ROWS_PER_BLOCK, E), lambda i: (i, 0))
            ],
            out_specs=[
                pl.BlockSpec((ROWS_PER_BLOCK, 128), lambda i: (i, 0)),
                pl.BlockSpec((ROWS_PER_BLOCK, 128), lambda i: (i, 0)),
            ],
            core_axis_name=("c", "s"),
            dimension_semantics=(pltpu.PARALLEL,),
        )(x_hbm, w_hbm, i_hbm)

    w_wide, i_wide = sc_run(x)
    return w_wide[:, :TOP_K], i_wide[:, :TOP_K]
